# Initial kernel scaffold; baseline (speedup 1.0000x reference)
import jax
import jax.numpy as jnp
from jax.experimental import pallas as pl


def kernel(x, edge_index, Wc0, bc0, Ws0, bs0, Wc1, bc1, Ws1, bs1, Wc2, bc2, Ws2, bs2, W1, W2, bn_gamma, bn_beta):
    raise NotImplementedError("write your pallas kernel here")


# speedup vs baseline: 6.2967x; 6.2967x over previous
import functools

import jax
import jax.numpy as jnp
from jax import lax
from jax.experimental import pallas as pl
from jax.experimental.pallas import tpu as pltpu
from jax.experimental.pallas import tpu_sc as plsc

N0 = 10000
E = 320000
DIN = 128
HID = 128
OUT = 16

NPAD = 10240
RR = NPAD // 128
DUMP = 10000
NC = 2
NS = 16
NW = NC * NS
CH = 128
PAD_E = 327680
G_FEAT = PAD_E // (NW * CH)
G_VEC = PAD_E // (NS * CH)
RPS = NPAD // NS

_MESH = plsc.VectorSubcoreMesh(core_axis_name="c", subcore_axis_name="s",
                               num_cores=NC, num_subcores=NS)


def _make_edge_pass(D):
  @functools.partial(
      pl.kernel,
      out_type=jax.ShapeDtypeStruct((NC, NPAD, D), jnp.float32),
      mesh=_MESH,
      compiler_params=pltpu.CompilerParams(use_tc_tiling_on_sc=(D == HID)),
      scratch_types=[
          pltpu.VMEM((CH,), jnp.int32),
          pltpu.VMEM((CH,), jnp.int32),
          pltpu.VMEM((CH, D), jnp.float32),
          pltpu.VMEM_SHARED((NPAD, D), jnp.float32),
          pltpu.SemaphoreType.DMA,
      ],
  )
  def kern(table_hbm, gidx_hbm, sidx_hbm, zeros_hbm, out_hbm,
           gv, sv, rows, acc, sem):
    c = lax.axis_index("c")
    s = lax.axis_index("s")
    tile = s * NC + c
    pltpu.sync_copy(zeros_hbm.at[pl.ds(s * RPS, RPS)],
                    acc.at[pl.ds(s * RPS, RPS)])
    plsc.subcore_barrier()

    def body(g, carry):
      base = tile * (G_FEAT * CH) + g * CH
      pltpu.sync_copy(gidx_hbm.at[pl.ds(base, CH)], gv)
      pltpu.sync_copy(sidx_hbm.at[pl.ds(base, CH)], sv)
      pltpu.async_copy(table_hbm.at[gv], rows, sem).wait()
      pltpu.sync_copy(rows, acc.at[sv], add=True)
      return carry

    lax.fori_loop(0, G_FEAT, body, 0)
    plsc.subcore_barrier()
    pltpu.sync_copy(acc.at[pl.ds(s * RPS, RPS)],
                    out_hbm.at[c, pl.ds(s * RPS, RPS)])

  return kern


_edge_pass_feat = _make_edge_pass(HID)
_edge_pass_8 = _make_edge_pass(8)


def _vec_pass(col, gidx, sidx, z8):
  tbl = jnp.pad(col, ((0, 0), (0, 7)))
  p = _edge_pass_8(tbl, gidx, sidx, z8)
  return p[0, :, 0:1] + p[1, :, 0:1]


def _mm_body(x_ref, w_ref, o_ref):
  o_ref[...] = jnp.dot(x_ref[...], w_ref[...],
                       preferred_element_type=jnp.float32)


def _tc_matmul(x, w):
  return pl.pallas_call(
      _mm_body,
      out_shape=jax.ShapeDtypeStruct((x.shape[0], w.shape[1]), jnp.float32),
  )(x, w)


def _rsq(deg):
  return lax.rsqrt(jnp.maximum(deg, 1.0))


def _scale_body(h_ref, dego_ref, o_ref):
  o_ref[...] = h_ref[...] * _rsq(dego_ref[...])


def _tc_scale(h, dego_c):
  return pl.pallas_call(
      _scale_body,
      out_shape=jax.ShapeDtypeStruct((NPAD, HID), jnp.float32),
  )(h, dego_c)


def _post_body(aggp_ref, degi_ref, dego_ref, bc_ref, ws_ref, selv_ref,
               out_ref, stbl_ref):
  agg = aggp_ref[0] + aggp_ref[1]
  o = jnp.maximum(agg * _rsq(degi_ref[...]) + bc_ref[...], 0.0)
  out_ref[...] = o
  sc = jnp.dot(o, ws_ref[...], preferred_element_type=jnp.float32)
  sc = sc * _rsq(dego_ref[...])
  stbl_ref[...] = jnp.where(selv_ref[...] > 0.0, sc, 0.0)


def _tc_post(aggp, degi_c, dego_c, bc, ws, selv_c):
  return pl.pallas_call(
      _post_body,
      out_shape=[
          jax.ShapeDtypeStruct((NPAD, HID), jnp.float32),
          jax.ShapeDtypeStruct((NPAD, 1), jnp.float32),
      ],
  )(aggp, degi_c, dego_c, bc, ws, selv_c)


def _score_key(sagg, degi, bs, validf):
  score = sagg * _rsq(degi) + bs
  mscore = jnp.where(validf > 0.0, score, -jnp.inf)
  u = lax.bitcast_convert_type(mscore, jnp.uint32)
  key = jnp.where((u >> 31) != 0, ~u, u | jnp.uint32(0x80000000))
  return score, key


def _select_body(k, saggd_ref, degid_ref, selvd_ref, saggc_ref, degic_ref,
                 selvc_ref, bs_ref, out_ref, feat_ref, selnew_ref, ro_ref):
  bs = bs_ref[0, 0]
  _, key = _score_key(saggd_ref[...], degid_ref[...], bs, selvd_ref[...])
  prefix = jnp.uint32(0)
  for b in range(31, -1, -1):
    cand = prefix | jnp.uint32(1 << b)
    cnt = jnp.sum((key >= cand).astype(jnp.int32))
    prefix = jnp.where(cnt >= k, cand, prefix)
  t = prefix
  cnt_gt = jnp.sum((key > t).astype(jnp.int32))
  need_eq = k - cnt_gt
  idx = lax.broadcasted_iota(jnp.int32, (RR, 128), 0) * 128 + \
      lax.broadcasted_iota(jnp.int32, (RR, 128), 1)
  eq = key == t
  cut = jnp.int32(0)
  for b in range(13, -1, -1):
    cand = cut | (1 << b)
    c = jnp.sum((eq & (idx < cand)).astype(jnp.int32))
    cut = jnp.where(c <= need_eq, cand, cut)

  score_c, key_c = _score_key(saggc_ref[...], degic_ref[...], bs,
                              selvc_ref[...])
  idx_c = lax.broadcasted_iota(jnp.int32, (NPAD, 1), 0)
  sel = (key_c > t) | ((key_c == t) & (idx_c < cut))
  selnew_ref[...] = sel.astype(jnp.float32)
  scaled = out_ref[...] * jnp.tanh(score_c)
  featn = jnp.where(sel, scaled, 0.0)
  feat_ref[...] = featn
  mean = jnp.sum(featn, axis=0) * (1.0 / k)
  mx = jnp.max(jnp.where(sel, scaled, -jnp.inf), axis=0)
  ro_ref[...] = jnp.concatenate([mean, mx])[None, :]


def _tc_select(sagg_c, degi_c, selv_c, bs, out, k):
  return pl.pallas_call(
      functools.partial(_select_body, k),
      out_shape=[
          jax.ShapeDtypeStruct((NPAD, HID), jnp.float32),
          jax.ShapeDtypeStruct((NPAD, 1), jnp.float32),
          jax.ShapeDtypeStruct((1, 2 * HID), jnp.float32),
      ],
  )(jnp.reshape(sagg_c, (RR, 128)), jnp.reshape(degi_c, (RR, 128)),
    jnp.reshape(selv_c, (RR, 128)), sagg_c, degi_c, selv_c, bs, out)


def _mlp_body(ro_ref, w1_ref, w2_ref, g_ref, b_ref, o_ref):
  h = jnp.dot(ro_ref[...], w1_ref[...], preferred_element_type=jnp.float32)
  h = h * (g_ref[...] * (1.0 + 1e-5) ** -0.5) + b_ref[...]
  h = jnp.maximum(h, 0.0)
  h = jnp.dot(h, w2_ref[...], preferred_element_type=jnp.float32)
  m = jnp.max(h, axis=-1, keepdims=True)
  z = h - m
  o_ref[...] = z - jnp.log(jnp.sum(jnp.exp(z), axis=-1, keepdims=True))


def _tc_mlp(ro, w1, w2, g, b):
  return pl.pallas_call(
      _mlp_body,
      out_shape=jax.ShapeDtypeStruct((1, OUT), jnp.float32),
  )(ro, w1, w2, g, b)


def kernel(x, edge_index, Wc0, bc0, Ws0, bs0, Wc1, bc1, Ws1, bs1,
           Wc2, bc2, Ws2, bs2, W1, W2, bn_gamma, bn_beta):
  src = edge_index[0].astype(jnp.int32)
  dst = edge_index[1].astype(jnp.int32)
  pad = PAD_E - E
  srcp = jnp.concatenate([src, jnp.zeros((pad,), jnp.int32)])
  dstp = jnp.concatenate([dst, jnp.full((pad,), DUMP, jnp.int32)])

  feat = jnp.zeros((NPAD, DIN), jnp.float32).at[:N0].set(x)
  selv = (lax.broadcasted_iota(jnp.int32, (NPAD, 1), 0) < N0
          ).astype(jnp.float32)
  z128 = jnp.zeros((NPAD, HID), jnp.float32)
  z8 = jnp.zeros((NPAD, 8), jnp.float32)

  Wcs = [Wc0, Wc1, Wc2]
  bcs = [bc0.reshape(1, HID), bc1.reshape(1, HID), bc2.reshape(1, HID)]
  Wss = [Ws0, Ws1, Ws2]
  bss = [bs0.reshape(1, 1), bs1.reshape(1, 1), bs2.reshape(1, 1)]
  ks = [5000, 2500, 1250]

  ro_total = None
  for i in range(3):
    dego_c = _vec_pass(selv, dstp, srcp, z8)
    degi_c = _vec_pass(selv, srcp, dstp, z8)
    h = _tc_matmul(feat, Wcs[i])
    h_scaled = _tc_scale(h, dego_c)
    agg_p = _edge_pass_feat(h_scaled, srcp, dstp, z128)
    out, s_tbl = _tc_post(agg_p, degi_c, dego_c, bcs[i], Wss[i], selv)
    sagg_c = _vec_pass(s_tbl, srcp, dstp, z8)
    feat, selv, ro = _tc_select(sagg_c, degi_c, selv, bss[i], out, ks[i])
    ro_total = ro if ro_total is None else ro_total + ro

  return _tc_mlp(ro_total, W1, W2, bn_gamma, bn_beta)


# speedup vs baseline: 8.7536x; 1.3902x over previous
import functools

import jax
import jax.numpy as jnp
from jax import lax
from jax.experimental import pallas as pl
from jax.experimental.pallas import tpu as pltpu
from jax.experimental.pallas import tpu_sc as plsc

N0 = 10000
E = 320000
DIN = 128
HID = 128
OUT = 16

NPAD = 10240
RR = NPAD // 128
DUMP = 10000
NC = 2
NS = 16
NW = NC * NS
CH = 128
PAD_E = 327680
G_FEAT = PAD_E // (NW * CH)
G_VEC = PAD_E // (NS * CH)
RPS = NPAD // NS

_MESH = plsc.VectorSubcoreMesh(core_axis_name="c", subcore_axis_name="s",
                               num_cores=NC, num_subcores=NS)


def _make_edge_pass(D):
  @functools.partial(
      pl.kernel,
      out_type=jax.ShapeDtypeStruct((NC, NPAD, D), jnp.float32),
      mesh=_MESH,
      compiler_params=pltpu.CompilerParams(use_tc_tiling_on_sc=(D == HID)),
      scratch_types=[
          pltpu.VMEM((CH,), jnp.int32),
          pltpu.VMEM((CH,), jnp.int32),
          pltpu.VMEM((CH,), jnp.int32),
          pltpu.VMEM((CH,), jnp.int32),
          pltpu.VMEM((CH, D), jnp.float32),
          pltpu.VMEM((CH, D), jnp.float32),
          pltpu.VMEM_SHARED((NPAD, D), jnp.float32),
          pltpu.SemaphoreType.DMA,
          pltpu.SemaphoreType.DMA,
      ],
  )
  def kern(table_hbm, gidx_hbm, sidx_hbm, zeros_hbm, out_hbm,
           gv0, gv1, sv0, sv1, rows0, rows1, acc, sem0, sem1):
    c = lax.axis_index("c")
    s = lax.axis_index("s")
    tile = s * NC + c
    pltpu.sync_copy(zeros_hbm.at[pl.ds(s * RPS, RPS)],
                    acc.at[pl.ds(s * RPS, RPS)])
    plsc.subcore_barrier()

    def body(i, carry):
      a = tile * (G_FEAT * CH) + (2 * i) * CH
      b = a + CH
      pltpu.sync_copy(gidx_hbm.at[pl.ds(a, CH)], gv0)
      cp0 = pltpu.async_copy(table_hbm.at[gv0], rows0, sem0)
      pltpu.sync_copy(gidx_hbm.at[pl.ds(b, CH)], gv1)
      cp1 = pltpu.async_copy(table_hbm.at[gv1], rows1, sem1)
      pltpu.sync_copy(sidx_hbm.at[pl.ds(a, CH)], sv0)
      pltpu.sync_copy(sidx_hbm.at[pl.ds(b, CH)], sv1)
      cp0.wait()
      pltpu.sync_copy(rows0, acc.at[sv0], add=True)
      cp1.wait()
      pltpu.sync_copy(rows1, acc.at[sv1], add=True)
      return carry

    lax.fori_loop(0, G_FEAT // 2, body, 0)
    plsc.subcore_barrier()
    pltpu.sync_copy(acc.at[pl.ds(s * RPS, RPS)],
                    out_hbm.at[c, pl.ds(s * RPS, RPS)])

  return kern


_edge_pass_feat = _make_edge_pass(HID)
_edge_pass_8 = _make_edge_pass(8)


@functools.partial(
    pl.kernel,
    out_type=[
        jax.ShapeDtypeStruct((NC, NPAD, 8), jnp.float32),
        jax.ShapeDtypeStruct((NC, NPAD, 8), jnp.float32),
    ],
    mesh=_MESH,
    compiler_params=pltpu.CompilerParams(use_tc_tiling_on_sc=False),
    scratch_types=[
        pltpu.VMEM((CH,), jnp.int32),
        pltpu.VMEM((CH,), jnp.int32),
        pltpu.VMEM((CH, 8), jnp.float32),
        pltpu.VMEM((CH, 8), jnp.float32),
        pltpu.VMEM_SHARED((NPAD, 8), jnp.float32),
        pltpu.VMEM_SHARED((NPAD, 8), jnp.float32),
        pltpu.SemaphoreType.DMA,
        pltpu.SemaphoreType.DMA,
    ],
)
def _deg_pass(table_hbm, src_hbm, dst_hbm, zeros_hbm, outo_hbm, outi_hbm,
              sv, dv, rows_o, rows_i, acc_o, acc_i, sem0, sem1):
  c = lax.axis_index("c")
  s = lax.axis_index("s")
  tile = s * NC + c
  pltpu.sync_copy(zeros_hbm.at[pl.ds(s * RPS, RPS)],
                  acc_o.at[pl.ds(s * RPS, RPS)])
  pltpu.sync_copy(zeros_hbm.at[pl.ds(s * RPS, RPS)],
                  acc_i.at[pl.ds(s * RPS, RPS)])
  plsc.subcore_barrier()

  def body(g, carry):
    base = tile * (G_FEAT * CH) + g * CH
    pltpu.sync_copy(src_hbm.at[pl.ds(base, CH)], sv)
    pltpu.sync_copy(dst_hbm.at[pl.ds(base, CH)], dv)
    cpo = pltpu.async_copy(table_hbm.at[dv], rows_o, sem0)
    cpi = pltpu.async_copy(table_hbm.at[sv], rows_i, sem1)
    cpo.wait()
    pltpu.sync_copy(rows_o, acc_o.at[sv], add=True)
    cpi.wait()
    pltpu.sync_copy(rows_i, acc_i.at[dv], add=True)
    return carry

  lax.fori_loop(0, G_FEAT, body, 0)
  plsc.subcore_barrier()
  pltpu.sync_copy(acc_o.at[pl.ds(s * RPS, RPS)],
                  outo_hbm.at[c, pl.ds(s * RPS, RPS)])
  pltpu.sync_copy(acc_i.at[pl.ds(s * RPS, RPS)],
                  outi_hbm.at[c, pl.ds(s * RPS, RPS)])


def _vec_pass(col, gidx, sidx, z8):
  tbl = jnp.pad(col, ((0, 0), (0, 7)))
  p = _edge_pass_8(tbl, gidx, sidx, z8)
  return p[0, :, 0:1] + p[1, :, 0:1]


def _mm_body(x_ref, w_ref, o_ref):
  o_ref[...] = jnp.dot(x_ref[...], w_ref[...],
                       preferred_element_type=jnp.float32)


def _tc_matmul(x, w):
  return pl.pallas_call(
      _mm_body,
      out_shape=jax.ShapeDtypeStruct((x.shape[0], w.shape[1]), jnp.float32),
  )(x, w)


def _rsq(deg):
  return lax.rsqrt(jnp.maximum(deg, 1.0))


def _scale_body(h_ref, dego_ref, o_ref):
  o_ref[...] = h_ref[...] * _rsq(dego_ref[...])


def _tc_scale(h, dego_c):
  return pl.pallas_call(
      _scale_body,
      out_shape=jax.ShapeDtypeStruct((NPAD, HID), jnp.float32),
  )(h, dego_c)


def _post_body(aggp_ref, degi_ref, dego_ref, bc_ref, ws_ref, selv_ref,
               out_ref, stbl_ref):
  agg = aggp_ref[0] + aggp_ref[1]
  o = jnp.maximum(agg * _rsq(degi_ref[...]) + bc_ref[...], 0.0)
  out_ref[...] = o
  sc = jnp.dot(o, ws_ref[...], preferred_element_type=jnp.float32)
  sc = sc * _rsq(dego_ref[...])
  stbl_ref[...] = jnp.where(selv_ref[...] > 0.0, sc, 0.0)


def _tc_post(aggp, degi_c, dego_c, bc, ws, selv_c):
  return pl.pallas_call(
      _post_body,
      out_shape=[
          jax.ShapeDtypeStruct((NPAD, HID), jnp.float32),
          jax.ShapeDtypeStruct((NPAD, 1), jnp.float32),
      ],
  )(aggp, degi_c, dego_c, bc, ws, selv_c)


def _score_key(sagg, degi, bs, validf):
  score = sagg * _rsq(degi) + bs
  mscore = jnp.where(validf > 0.0, score, -jnp.inf)
  u = lax.bitcast_convert_type(mscore, jnp.uint32)
  key = jnp.where((u >> 31) != 0, ~u, u | jnp.uint32(0x80000000))
  return score, key


def _select_body(k, saggd_ref, degid_ref, selvd_ref, saggc_ref, degic_ref,
                 selvc_ref, bs_ref, out_ref, feat_ref, selnew_ref, ro_ref):
  bs = bs_ref[0, 0]
  _, key = _score_key(saggd_ref[...], degid_ref[...], bs, selvd_ref[...])
  prefix = jnp.uint32(0)
  for b in range(31, -1, -1):
    cand = prefix | jnp.uint32(1 << b)
    cnt = jnp.sum((key >= cand).astype(jnp.int32))
    prefix = jnp.where(cnt >= k, cand, prefix)
  t = prefix
  cnt_gt = jnp.sum((key > t).astype(jnp.int32))
  need_eq = k - cnt_gt
  idx = lax.broadcasted_iota(jnp.int32, (RR, 128), 0) * 128 + \
      lax.broadcasted_iota(jnp.int32, (RR, 128), 1)
  eq = key == t
  cut = jnp.int32(0)
  for b in range(13, -1, -1):
    cand = cut | (1 << b)
    c = jnp.sum((eq & (idx < cand)).astype(jnp.int32))
    cut = jnp.where(c <= need_eq, cand, cut)

  score_c, key_c = _score_key(saggc_ref[...], degic_ref[...], bs,
                              selvc_ref[...])
  idx_c = lax.broadcasted_iota(jnp.int32, (NPAD, 1), 0)
  sel = (key_c > t) | ((key_c == t) & (idx_c < cut))
  selnew_ref[...] = sel.astype(jnp.float32)
  scaled = out_ref[...] * jnp.tanh(score_c)
  featn = jnp.where(sel, scaled, 0.0)
  feat_ref[...] = featn
  mean = jnp.sum(featn, axis=0) * (1.0 / k)
  mx = jnp.max(jnp.where(sel, scaled, -jnp.inf), axis=0)
  ro_ref[...] = jnp.concatenate([mean, mx])[None, :]


def _tc_select(sagg_c, degi_c, selv_c, bs, out, k):
  return pl.pallas_call(
      functools.partial(_select_body, k),
      out_shape=[
          jax.ShapeDtypeStruct((NPAD, HID), jnp.float32),
          jax.ShapeDtypeStruct((NPAD, 1), jnp.float32),
          jax.ShapeDtypeStruct((1, 2 * HID), jnp.float32),
      ],
  )(jnp.reshape(sagg_c, (RR, 128)), jnp.reshape(degi_c, (RR, 128)),
    jnp.reshape(selv_c, (RR, 128)), sagg_c, degi_c, selv_c, bs, out)


def _mlp_body(ro_ref, w1_ref, w2_ref, g_ref, b_ref, o_ref):
  h = jnp.dot(ro_ref[...], w1_ref[...], preferred_element_type=jnp.float32)
  h = h * (g_ref[...] * (1.0 + 1e-5) ** -0.5) + b_ref[...]
  h = jnp.maximum(h, 0.0)
  h = jnp.dot(h, w2_ref[...], preferred_element_type=jnp.float32)
  m = jnp.max(h, axis=-1, keepdims=True)
  z = h - m
  o_ref[...] = z - jnp.log(jnp.sum(jnp.exp(z), axis=-1, keepdims=True))


def _tc_mlp(ro, w1, w2, g, b):
  return pl.pallas_call(
      _mlp_body,
      out_shape=jax.ShapeDtypeStruct((1, OUT), jnp.float32),
  )(ro, w1, w2, g, b)


def kernel(x, edge_index, Wc0, bc0, Ws0, bs0, Wc1, bc1, Ws1, bs1,
           Wc2, bc2, Ws2, bs2, W1, W2, bn_gamma, bn_beta):
  src = edge_index[0].astype(jnp.int32)
  dst = edge_index[1].astype(jnp.int32)
  pad = PAD_E - E
  srcp = jnp.concatenate([src, jnp.zeros((pad,), jnp.int32)])
  dstp = jnp.concatenate([dst, jnp.full((pad,), DUMP, jnp.int32)])

  feat = jnp.zeros((NPAD, DIN), jnp.float32).at[:N0].set(x)
  selv = (lax.broadcasted_iota(jnp.int32, (NPAD, 1), 0) < N0
          ).astype(jnp.float32)
  z128 = jnp.zeros((NPAD, HID), jnp.float32)
  z8 = jnp.zeros((NPAD, 8), jnp.float32)

  Wcs = [Wc0, Wc1, Wc2]
  bcs = [bc0.reshape(1, HID), bc1.reshape(1, HID), bc2.reshape(1, HID)]
  Wss = [Ws0, Ws1, Ws2]
  bss = [bs0.reshape(1, 1), bs1.reshape(1, 1), bs2.reshape(1, 1)]
  ks = [5000, 2500, 1250]

  ro_total = None
  for i in range(3):
    po, pi = _deg_pass(jnp.pad(selv, ((0, 0), (0, 7))), srcp, dstp, z8)
    dego_c = po[0, :, 0:1] + po[1, :, 0:1]
    degi_c = pi[0, :, 0:1] + pi[1, :, 0:1]
    h = _tc_matmul(feat, Wcs[i])
    h_scaled = _tc_scale(h, dego_c)
    agg_p = _edge_pass_feat(h_scaled, srcp, dstp, z128)
    out, s_tbl = _tc_post(agg_p, degi_c, dego_c, bcs[i], Wss[i], selv)
    sagg_c = _vec_pass(s_tbl, srcp, dstp, z8)
    feat, selv, ro = _tc_select(sagg_c, degi_c, selv, bss[i], out, ks[i])
    ro_total = ro if ro_total is None else ro_total + ro

  return _tc_mlp(ro_total, W1, W2, bn_gamma, bn_beta)


# speedup vs baseline: 10.1803x; 1.1630x over previous
import functools

import jax
import jax.numpy as jnp
from jax import lax
from jax.experimental import pallas as pl
from jax.experimental.pallas import tpu as pltpu
from jax.experimental.pallas import tpu_sc as plsc

N0 = 10000
E = 320000
DIN = 128
HID = 128
OUT = 16

NPAD = 10240
RR = NPAD // 128
DUMP = 10000
NC = 2
NS = 16
NW = NC * NS
CH = 128
PAD_E = 327680
G_FEAT = PAD_E // (NW * CH)
G_VEC = PAD_E // (NS * CH)
RPS = NPAD // NS

_MESH = plsc.VectorSubcoreMesh(core_axis_name="c", subcore_axis_name="s",
                               num_cores=NC, num_subcores=NS)


def _make_edge_pass(D):
  nbuf = 4 if D == 8 else 2
  preload_g = D == 8

  @functools.partial(
      pl.kernel,
      out_type=jax.ShapeDtypeStruct((NC, NPAD, D), jnp.float32),
      mesh=_MESH,
      compiler_params=pltpu.CompilerParams(use_tc_tiling_on_sc=(D == HID)),
      scratch_types=[
          (pltpu.VMEM((G_FEAT, CH), jnp.int32) if preload_g
           else [pltpu.VMEM((CH,), jnp.int32)] * nbuf),
          pltpu.VMEM((G_FEAT, CH), jnp.int32),
          [pltpu.VMEM((CH, D), jnp.float32)] * nbuf,
          pltpu.VMEM_SHARED((NPAD, D), jnp.float32),
          [pltpu.SemaphoreType.DMA] * nbuf,
          [pltpu.SemaphoreType.DMA] * nbuf,
      ],
  )
  def kern(table_hbm, gidx_hbm, sidx_hbm, zeros_hbm, out_hbm,
           gbuf, sidx_v, rows, acc, gsem, ssem):
    c = lax.axis_index("c")
    s = lax.axis_index("s")
    tile = s * NC + c
    pltpu.sync_copy(zeros_hbm.at[pl.ds(s * RPS, RPS)],
                    acc.at[pl.ds(s * RPS, RPS)])
    pltpu.sync_copy(sidx_hbm.at[pl.ds(tile * G_FEAT, G_FEAT)], sidx_v)
    if preload_g:
      pltpu.sync_copy(gidx_hbm.at[pl.ds(tile * G_FEAT, G_FEAT)], gbuf)
    plsc.subcore_barrier()

    def body(i, carry):
      q = i * nbuf
      cps = []
      for j in range(nbuf):
        if preload_g:
          gv = gbuf.at[q + j]
        else:
          gv = gbuf[j]
          pltpu.sync_copy(gidx_hbm.at[tile * G_FEAT + q + j], gv)
        cps.append(pltpu.async_copy(table_hbm.at[gv], rows[j], gsem[j]))
      scs = []
      for j in range(nbuf):
        cps[j].wait()
        scs.append(pltpu.async_copy(rows[j], acc.at[sidx_v.at[q + j]],
                                    ssem[j], add=True))
      for sc in scs:
        sc.wait()
      return carry

    lax.fori_loop(0, G_FEAT // nbuf, body, 0)
    plsc.subcore_barrier()
    pltpu.sync_copy(acc.at[pl.ds(s * RPS, RPS)],
                    out_hbm.at[c, pl.ds(s * RPS, RPS)])

  return kern


_edge_pass_feat = _make_edge_pass(HID)
_edge_pass_8 = _make_edge_pass(8)


@functools.partial(
    pl.kernel,
    out_type=[
        jax.ShapeDtypeStruct((NC, NPAD, 8), jnp.float32),
        jax.ShapeDtypeStruct((NC, NPAD, 8), jnp.float32),
    ],
    mesh=_MESH,
    compiler_params=pltpu.CompilerParams(use_tc_tiling_on_sc=False),
    scratch_types=[
        pltpu.VMEM((G_FEAT, CH), jnp.int32),
        pltpu.VMEM((G_FEAT, CH), jnp.int32),
        [pltpu.VMEM((CH, 8), jnp.float32)] * 4,
        pltpu.VMEM_SHARED((NPAD, 8), jnp.float32),
        pltpu.VMEM_SHARED((NPAD, 8), jnp.float32),
        [pltpu.SemaphoreType.DMA] * 4,
        [pltpu.SemaphoreType.DMA] * 4,
    ],
)
def _deg_pass(table_hbm, src_hbm, dst_hbm, zeros_hbm, outo_hbm, outi_hbm,
              sidx_v, didx_v, rows, acc_o, acc_i, gsem, ssem):
  c = lax.axis_index("c")
  s = lax.axis_index("s")
  tile = s * NC + c
  pltpu.sync_copy(zeros_hbm.at[pl.ds(s * RPS, RPS)],
                  acc_o.at[pl.ds(s * RPS, RPS)])
  pltpu.sync_copy(zeros_hbm.at[pl.ds(s * RPS, RPS)],
                  acc_i.at[pl.ds(s * RPS, RPS)])
  pltpu.sync_copy(src_hbm.at[pl.ds(tile * G_FEAT, G_FEAT)], sidx_v)
  pltpu.sync_copy(dst_hbm.at[pl.ds(tile * G_FEAT, G_FEAT)], didx_v)
  plsc.subcore_barrier()

  def body(i, carry):
    cps, scs = [], []
    for p in range(2):
      q = 2 * i + p
      cps.append(pltpu.async_copy(table_hbm.at[didx_v.at[q]], rows[2 * p],
                                  gsem[2 * p]))
      cps.append(pltpu.async_copy(table_hbm.at[sidx_v.at[q]], rows[2 * p + 1],
                                  gsem[2 * p + 1]))
    for p in range(2):
      q = 2 * i + p
      cps[2 * p].wait()
      scs.append(pltpu.async_copy(rows[2 * p], acc_o.at[sidx_v.at[q]],
                                  ssem[2 * p], add=True))
      cps[2 * p + 1].wait()
      scs.append(pltpu.async_copy(rows[2 * p + 1], acc_i.at[didx_v.at[q]],
                                  ssem[2 * p + 1], add=True))
    for sc in scs:
      sc.wait()
    return carry

  lax.fori_loop(0, G_FEAT // 2, body, 0)
  plsc.subcore_barrier()
  pltpu.sync_copy(acc_o.at[pl.ds(s * RPS, RPS)],
                  outo_hbm.at[c, pl.ds(s * RPS, RPS)])
  pltpu.sync_copy(acc_i.at[pl.ds(s * RPS, RPS)],
                  outi_hbm.at[c, pl.ds(s * RPS, RPS)])


def _vec_pass(col, gidx, sidx, z8):
  tbl = jnp.pad(col, ((0, 0), (0, 7)))
  p = _edge_pass_8(tbl, gidx, sidx, z8)
  return p[0, :, 0:1] + p[1, :, 0:1]


def _mm_body(x_ref, w_ref, o_ref):
  o_ref[...] = jnp.dot(x_ref[...], w_ref[...],
                       preferred_element_type=jnp.float32)


def _tc_matmul(x, w):
  return pl.pallas_call(
      _mm_body,
      out_shape=jax.ShapeDtypeStruct((x.shape[0], w.shape[1]), jnp.float32),
  )(x, w)


def _rsq(deg):
  return lax.rsqrt(jnp.maximum(deg, 1.0))


def _scale_body(h_ref, dego_ref, o_ref):
  o_ref[...] = h_ref[...] * _rsq(dego_ref[...])


def _tc_scale(h, dego_c):
  return pl.pallas_call(
      _scale_body,
      out_shape=jax.ShapeDtypeStruct((NPAD, HID), jnp.float32),
  )(h, dego_c)


def _post_body(aggp_ref, degi_ref, dego_ref, bc_ref, ws_ref, selv_ref,
               out_ref, stbl_ref):
  agg = aggp_ref[0] + aggp_ref[1]
  o = jnp.maximum(agg * _rsq(degi_ref[...]) + bc_ref[...], 0.0)
  out_ref[...] = o
  sc = jnp.dot(o, ws_ref[...], preferred_element_type=jnp.float32)
  sc = sc * _rsq(dego_ref[...])
  stbl_ref[...] = jnp.where(selv_ref[...] > 0.0, sc, 0.0)


def _tc_post(aggp, degi_c, dego_c, bc, ws, selv_c):
  return pl.pallas_call(
      _post_body,
      out_shape=[
          jax.ShapeDtypeStruct((NPAD, HID), jnp.float32),
          jax.ShapeDtypeStruct((NPAD, 1), jnp.float32),
      ],
  )(aggp, degi_c, dego_c, bc, ws, selv_c)


def _score_key(sagg, degi, bs, validf):
  score = sagg * _rsq(degi) + bs
  mscore = jnp.where(validf > 0.0, score, -jnp.inf)
  u = lax.bitcast_convert_type(mscore, jnp.uint32)
  key = jnp.where((u >> 31) != 0, ~u, u | jnp.uint32(0x80000000))
  return score, key


def _select_body(k, saggd_ref, degid_ref, selvd_ref, saggc_ref, degic_ref,
                 selvc_ref, bs_ref, out_ref, feat_ref, selnew_ref, ro_ref):
  bs = bs_ref[0, 0]
  _, key = _score_key(saggd_ref[...], degid_ref[...], bs, selvd_ref[...])
  prefix = jnp.uint32(0)
  for b in range(31, -1, -1):
    cand = prefix | jnp.uint32(1 << b)
    cnt = jnp.sum((key >= cand).astype(jnp.int32))
    prefix = jnp.where(cnt >= k, cand, prefix)
  t = prefix
  cnt_gt = jnp.sum((key > t).astype(jnp.int32))
  need_eq = k - cnt_gt
  idx = lax.broadcasted_iota(jnp.int32, (RR, 128), 0) * 128 + \
      lax.broadcasted_iota(jnp.int32, (RR, 128), 1)
  eq = key == t
  cut = jnp.int32(0)
  for b in range(13, -1, -1):
    cand = cut | (1 << b)
    c = jnp.sum((eq & (idx < cand)).astype(jnp.int32))
    cut = jnp.where(c <= need_eq, cand, cut)

  score_c, key_c = _score_key(saggc_ref[...], degic_ref[...], bs,
                              selvc_ref[...])
  idx_c = lax.broadcasted_iota(jnp.int32, (NPAD, 1), 0)
  sel = (key_c > t) | ((key_c == t) & (idx_c < cut))
  selnew_ref[...] = sel.astype(jnp.float32)
  scaled = out_ref[...] * jnp.tanh(score_c)
  featn = jnp.where(sel, scaled, 0.0)
  feat_ref[...] = featn
  mean = jnp.sum(featn, axis=0) * (1.0 / k)
  mx = jnp.max(jnp.where(sel, scaled, -jnp.inf), axis=0)
  ro_ref[...] = jnp.concatenate([mean, mx])[None, :]


def _tc_select(sagg_c, degi_c, selv_c, bs, out, k):
  return pl.pallas_call(
      functools.partial(_select_body, k),
      out_shape=[
          jax.ShapeDtypeStruct((NPAD, HID), jnp.float32),
          jax.ShapeDtypeStruct((NPAD, 1), jnp.float32),
          jax.ShapeDtypeStruct((1, 2 * HID), jnp.float32),
      ],
  )(jnp.reshape(sagg_c, (RR, 128)), jnp.reshape(degi_c, (RR, 128)),
    jnp.reshape(selv_c, (RR, 128)), sagg_c, degi_c, selv_c, bs, out)


def _mlp_body(ro_ref, w1_ref, w2_ref, g_ref, b_ref, o_ref):
  h = jnp.dot(ro_ref[...], w1_ref[...], preferred_element_type=jnp.float32)
  h = h * (g_ref[...] * (1.0 + 1e-5) ** -0.5) + b_ref[...]
  h = jnp.maximum(h, 0.0)
  h = jnp.dot(h, w2_ref[...], preferred_element_type=jnp.float32)
  m = jnp.max(h, axis=-1, keepdims=True)
  z = h - m
  o_ref[...] = z - jnp.log(jnp.sum(jnp.exp(z), axis=-1, keepdims=True))


def _tc_mlp(ro, w1, w2, g, b):
  return pl.pallas_call(
      _mlp_body,
      out_shape=jax.ShapeDtypeStruct((1, OUT), jnp.float32),
  )(ro, w1, w2, g, b)


def kernel(x, edge_index, Wc0, bc0, Ws0, bs0, Wc1, bc1, Ws1, bs1,
           Wc2, bc2, Ws2, bs2, W1, W2, bn_gamma, bn_beta):
  src = edge_index[0].astype(jnp.int32)
  dst = edge_index[1].astype(jnp.int32)
  pad = PAD_E - E
  srcp = jnp.concatenate([src, jnp.zeros((pad,), jnp.int32)]
                         ).reshape(PAD_E // CH, CH)
  dstp = jnp.concatenate([dst, jnp.full((pad,), DUMP, jnp.int32)]
                         ).reshape(PAD_E // CH, CH)

  feat = jnp.zeros((NPAD, DIN), jnp.float32).at[:N0].set(x)
  selv = (lax.broadcasted_iota(jnp.int32, (NPAD, 1), 0) < N0
          ).astype(jnp.float32)
  z128 = jnp.zeros((NPAD, HID), jnp.float32)
  z8 = jnp.zeros((NPAD, 8), jnp.float32)

  Wcs = [Wc0, Wc1, Wc2]
  bcs = [bc0.reshape(1, HID), bc1.reshape(1, HID), bc2.reshape(1, HID)]
  Wss = [Ws0, Ws1, Ws2]
  bss = [bs0.reshape(1, 1), bs1.reshape(1, 1), bs2.reshape(1, 1)]
  ks = [5000, 2500, 1250]

  ro_total = None
  for i in range(3):
    po, pi = _deg_pass(jnp.pad(selv, ((0, 0), (0, 7))), srcp, dstp, z8)
    dego_c = po[0, :, 0:1] + po[1, :, 0:1]
    degi_c = pi[0, :, 0:1] + pi[1, :, 0:1]
    h = _tc_matmul(feat, Wcs[i])
    h_scaled = _tc_scale(h, dego_c)
    agg_p = _edge_pass_feat(h_scaled, srcp, dstp, z128)
    out, s_tbl = _tc_post(agg_p, degi_c, dego_c, bcs[i], Wss[i], selv)
    sagg_c = _vec_pass(s_tbl, srcp, dstp, z8)
    feat, selv, ro = _tc_select(sagg_c, degi_c, selv, bss[i], out, ks[i])
    ro_total = ro if ro_total is None else ro_total + ro

  return _tc_mlp(ro_total, W1, W2, bn_gamma, bn_beta)


# speedup vs baseline: 10.2658x; 1.0084x over previous
import functools

import jax
import jax.numpy as jnp
from jax import lax
from jax.experimental import pallas as pl
from jax.experimental.pallas import tpu as pltpu
from jax.experimental.pallas import tpu_sc as plsc

N0 = 10000
E = 320000
DIN = 128
HID = 128
OUT = 16

NPAD = 10240
RR = NPAD // 128
DUMP = 10000
NC = 2
NS = 16
NW = NC * NS
CH = 128
PAD_E = 327680
G_FEAT = PAD_E // (NW * CH)
G_VEC = PAD_E // (NS * CH)
RPS = NPAD // NS

_MESH = plsc.VectorSubcoreMesh(core_axis_name="c", subcore_axis_name="s",
                               num_cores=NC, num_subcores=NS)


def _make_edge_pass(D):
  nbuf = 8 if D == 8 else 2
  preload_g = D == 8

  @functools.partial(
      pl.kernel,
      out_type=jax.ShapeDtypeStruct((NC, NPAD, D), jnp.float32),
      mesh=_MESH,
      compiler_params=pltpu.CompilerParams(use_tc_tiling_on_sc=(D == HID)),
      scratch_types=[
          (pltpu.VMEM((G_FEAT, CH), jnp.int32) if preload_g
           else [pltpu.VMEM((CH,), jnp.int32)] * nbuf),
          pltpu.VMEM((G_FEAT, CH), jnp.int32),
          [pltpu.VMEM((CH, D), jnp.float32)] * nbuf,
          pltpu.VMEM_SHARED((NPAD, D), jnp.float32),
          [pltpu.SemaphoreType.DMA] * nbuf,
          [pltpu.SemaphoreType.DMA] * nbuf,
      ],
  )
  def kern(table_hbm, gidx_hbm, sidx_hbm, zeros_hbm, out_hbm,
           gbuf, sidx_v, rows, acc, gsem, ssem):
    c = lax.axis_index("c")
    s = lax.axis_index("s")
    tile = s * NC + c
    pltpu.sync_copy(zeros_hbm.at[pl.ds(s * RPS, RPS)],
                    acc.at[pl.ds(s * RPS, RPS)])
    pltpu.sync_copy(sidx_hbm.at[pl.ds(tile * G_FEAT, G_FEAT)], sidx_v)
    if preload_g:
      pltpu.sync_copy(gidx_hbm.at[pl.ds(tile * G_FEAT, G_FEAT)], gbuf)
    plsc.subcore_barrier()

    def body(i, carry):
      q = i * nbuf
      cps = []
      for j in range(nbuf):
        if preload_g:
          gv = gbuf.at[q + j]
        else:
          gv = gbuf[j]
          pltpu.sync_copy(gidx_hbm.at[tile * G_FEAT + q + j], gv)
        cps.append(pltpu.async_copy(table_hbm.at[gv], rows[j], gsem[j]))
      scs = []
      for j in range(nbuf):
        cps[j].wait()
        scs.append(pltpu.async_copy(rows[j], acc.at[sidx_v.at[q + j]],
                                    ssem[j], add=True))
      for sc in scs:
        sc.wait()
      return carry

    lax.fori_loop(0, G_FEAT // nbuf, body, 0)
    plsc.subcore_barrier()
    pltpu.sync_copy(acc.at[pl.ds(s * RPS, RPS)],
                    out_hbm.at[c, pl.ds(s * RPS, RPS)])

  return kern


_edge_pass_feat = _make_edge_pass(HID)
_edge_pass_8 = _make_edge_pass(8)


@functools.partial(
    pl.kernel,
    out_type=[
        jax.ShapeDtypeStruct((NC, NPAD, 8), jnp.float32),
        jax.ShapeDtypeStruct((NC, NPAD, 8), jnp.float32),
    ],
    mesh=_MESH,
    compiler_params=pltpu.CompilerParams(use_tc_tiling_on_sc=False),
    scratch_types=[
        pltpu.VMEM((G_FEAT, CH), jnp.int32),
        pltpu.VMEM((G_FEAT, CH), jnp.int32),
        [pltpu.VMEM((CH, 8), jnp.float32)] * 4,
        pltpu.VMEM_SHARED((NPAD, 8), jnp.float32),
        pltpu.VMEM_SHARED((NPAD, 8), jnp.float32),
        [pltpu.SemaphoreType.DMA] * 4,
        [pltpu.SemaphoreType.DMA] * 4,
    ],
)
def _deg_pass(table_hbm, src_hbm, dst_hbm, zeros_hbm, outo_hbm, outi_hbm,
              sidx_v, didx_v, rows, acc_o, acc_i, gsem, ssem):
  c = lax.axis_index("c")
  s = lax.axis_index("s")
  tile = s * NC + c
  pltpu.sync_copy(zeros_hbm.at[pl.ds(s * RPS, RPS)],
                  acc_o.at[pl.ds(s * RPS, RPS)])
  pltpu.sync_copy(zeros_hbm.at[pl.ds(s * RPS, RPS)],
                  acc_i.at[pl.ds(s * RPS, RPS)])
  pltpu.sync_copy(src_hbm.at[pl.ds(tile * G_FEAT, G_FEAT)], sidx_v)
  pltpu.sync_copy(dst_hbm.at[pl.ds(tile * G_FEAT, G_FEAT)], didx_v)
  plsc.subcore_barrier()

  def body(i, carry):
    cps, scs = [], []
    for p in range(2):
      q = 2 * i + p
      cps.append(pltpu.async_copy(table_hbm.at[didx_v.at[q]], rows[2 * p],
                                  gsem[2 * p]))
      cps.append(pltpu.async_copy(table_hbm.at[sidx_v.at[q]], rows[2 * p + 1],
                                  gsem[2 * p + 1]))
    for p in range(2):
      q = 2 * i + p
      cps[2 * p].wait()
      scs.append(pltpu.async_copy(rows[2 * p], acc_o.at[sidx_v.at[q]],
                                  ssem[2 * p], add=True))
      cps[2 * p + 1].wait()
      scs.append(pltpu.async_copy(rows[2 * p + 1], acc_i.at[didx_v.at[q]],
                                  ssem[2 * p + 1], add=True))
    for sc in scs:
      sc.wait()
    return carry

  lax.fori_loop(0, G_FEAT // 2, body, 0)
  plsc.subcore_barrier()
  pltpu.sync_copy(acc_o.at[pl.ds(s * RPS, RPS)],
                  outo_hbm.at[c, pl.ds(s * RPS, RPS)])
  pltpu.sync_copy(acc_i.at[pl.ds(s * RPS, RPS)],
                  outi_hbm.at[c, pl.ds(s * RPS, RPS)])


def _vec_pass(col, gidx, sidx, z8):
  tbl = jnp.pad(col, ((0, 0), (0, 7)))
  p = _edge_pass_8(tbl, gidx, sidx, z8)
  return p[0, :, 0:1] + p[1, :, 0:1]


def _mm_body(x_ref, w_ref, o_ref):
  o_ref[...] = jnp.dot(x_ref[...], w_ref[...],
                       preferred_element_type=jnp.float32)


def _tc_matmul(x, w):
  return pl.pallas_call(
      _mm_body,
      out_shape=jax.ShapeDtypeStruct((x.shape[0], w.shape[1]), jnp.float32),
  )(x, w)


def _rsq(deg):
  return lax.rsqrt(jnp.maximum(deg, 1.0))


def _scale_body(h_ref, dego_ref, o_ref):
  o_ref[...] = h_ref[...] * _rsq(dego_ref[...])


def _tc_scale(h, dego_c):
  return pl.pallas_call(
      _scale_body,
      out_shape=jax.ShapeDtypeStruct((NPAD, HID), jnp.float32),
  )(h, dego_c)


def _post_body(aggp_ref, degi_ref, dego_ref, bc_ref, ws_ref, selv_ref,
               out_ref, stbl_ref):
  agg = aggp_ref[0] + aggp_ref[1]
  o = jnp.maximum(agg * _rsq(degi_ref[...]) + bc_ref[...], 0.0)
  out_ref[...] = o
  sc = jnp.dot(o, ws_ref[...], preferred_element_type=jnp.float32)
  sc = sc * _rsq(dego_ref[...])
  stbl_ref[...] = jnp.where(selv_ref[...] > 0.0, sc, 0.0)


def _tc_post(aggp, degi_c, dego_c, bc, ws, selv_c):
  return pl.pallas_call(
      _post_body,
      out_shape=[
          jax.ShapeDtypeStruct((NPAD, HID), jnp.float32),
          jax.ShapeDtypeStruct((NPAD, 1), jnp.float32),
      ],
  )(aggp, degi_c, dego_c, bc, ws, selv_c)


def _score_key(sagg, degi, bs, validf):
  score = sagg * _rsq(degi) + bs
  mscore = jnp.where(validf > 0.0, score, -jnp.inf)
  u = lax.bitcast_convert_type(mscore, jnp.uint32)
  key = jnp.where((u >> 31) != 0, ~u, u | jnp.uint32(0x80000000))
  return score, key


def _select_body(k, saggd_ref, degid_ref, selvd_ref, saggc_ref, degic_ref,
                 selvc_ref, bs_ref, out_ref, feat_ref, selnew_ref, ro_ref):
  bs = bs_ref[0, 0]
  _, key = _score_key(saggd_ref[...], degid_ref[...], bs, selvd_ref[...])
  prefix = jnp.uint32(0)
  for b in range(31, -1, -1):
    cand = prefix | jnp.uint32(1 << b)
    cnt = jnp.sum((key >= cand).astype(jnp.int32))
    prefix = jnp.where(cnt >= k, cand, prefix)
  t = prefix
  cnt_gt = jnp.sum((key > t).astype(jnp.int32))
  need_eq = k - cnt_gt
  idx = lax.broadcasted_iota(jnp.int32, (RR, 128), 0) * 128 + \
      lax.broadcasted_iota(jnp.int32, (RR, 128), 1)
  eq = key == t
  cut = jnp.int32(0)
  for b in range(13, -1, -1):
    cand = cut | (1 << b)
    c = jnp.sum((eq & (idx < cand)).astype(jnp.int32))
    cut = jnp.where(c <= need_eq, cand, cut)

  score_c, key_c = _score_key(saggc_ref[...], degic_ref[...], bs,
                              selvc_ref[...])
  idx_c = lax.broadcasted_iota(jnp.int32, (NPAD, 1), 0)
  sel = (key_c > t) | ((key_c == t) & (idx_c < cut))
  selnew_ref[...] = sel.astype(jnp.float32)
  scaled = out_ref[...] * jnp.tanh(score_c)
  featn = jnp.where(sel, scaled, 0.0)
  feat_ref[...] = featn
  mean = jnp.sum(featn, axis=0) * (1.0 / k)
  mx = jnp.max(jnp.where(sel, scaled, -jnp.inf), axis=0)
  ro_ref[...] = jnp.concatenate([mean, mx])[None, :]


def _tc_select(sagg_c, degi_c, selv_c, bs, out, k):
  return pl.pallas_call(
      functools.partial(_select_body, k),
      out_shape=[
          jax.ShapeDtypeStruct((NPAD, HID), jnp.float32),
          jax.ShapeDtypeStruct((NPAD, 1), jnp.float32),
          jax.ShapeDtypeStruct((1, 2 * HID), jnp.float32),
      ],
  )(jnp.reshape(sagg_c, (RR, 128)), jnp.reshape(degi_c, (RR, 128)),
    jnp.reshape(selv_c, (RR, 128)), sagg_c, degi_c, selv_c, bs, out)


def _mlp_body(ro_ref, w1_ref, w2_ref, g_ref, b_ref, o_ref):
  h = jnp.dot(ro_ref[...], w1_ref[...], preferred_element_type=jnp.float32)
  h = h * (g_ref[...] * (1.0 + 1e-5) ** -0.5) + b_ref[...]
  h = jnp.maximum(h, 0.0)
  h = jnp.dot(h, w2_ref[...], preferred_element_type=jnp.float32)
  m = jnp.max(h, axis=-1, keepdims=True)
  z = h - m
  o_ref[...] = z - jnp.log(jnp.sum(jnp.exp(z), axis=-1, keepdims=True))


def _tc_mlp(ro, w1, w2, g, b):
  return pl.pallas_call(
      _mlp_body,
      out_shape=jax.ShapeDtypeStruct((1, OUT), jnp.float32),
  )(ro, w1, w2, g, b)


def kernel(x, edge_index, Wc0, bc0, Ws0, bs0, Wc1, bc1, Ws1, bs1,
           Wc2, bc2, Ws2, bs2, W1, W2, bn_gamma, bn_beta):
  src = edge_index[0].astype(jnp.int32)
  dst = edge_index[1].astype(jnp.int32)
  pad = PAD_E - E
  srcp = jnp.concatenate([src, jnp.zeros((pad,), jnp.int32)]
                         ).reshape(PAD_E // CH, CH)
  dstp = jnp.concatenate([dst, jnp.full((pad,), DUMP, jnp.int32)]
                         ).reshape(PAD_E // CH, CH)

  feat = jnp.zeros((NPAD, DIN), jnp.float32).at[:N0].set(x)
  selv = (lax.broadcasted_iota(jnp.int32, (NPAD, 1), 0) < N0
          ).astype(jnp.float32)
  z128 = jnp.zeros((NPAD, HID), jnp.float32)
  z8 = jnp.zeros((NPAD, 8), jnp.float32)

  Wcs = [Wc0, Wc1, Wc2]
  bcs = [bc0.reshape(1, HID), bc1.reshape(1, HID), bc2.reshape(1, HID)]
  Wss = [Ws0, Ws1, Ws2]
  bss = [bs0.reshape(1, 1), bs1.reshape(1, 1), bs2.reshape(1, 1)]
  ks = [5000, 2500, 1250]

  ro_total = None
  for i in range(3):
    po, pi = _deg_pass(jnp.pad(selv, ((0, 0), (0, 7))), srcp, dstp, z8)
    dego_c = po[0, :, 0:1] + po[1, :, 0:1]
    degi_c = pi[0, :, 0:1] + pi[1, :, 0:1]
    h = _tc_matmul(feat, Wcs[i])
    h_scaled = _tc_scale(h, dego_c)
    agg_p = _edge_pass_feat(h_scaled, srcp, dstp, z128)
    out, s_tbl = _tc_post(agg_p, degi_c, dego_c, bcs[i], Wss[i], selv)
    sagg_c = _vec_pass(s_tbl, srcp, dstp, z8)
    feat, selv, ro = _tc_select(sagg_c, degi_c, selv, bss[i], out, ks[i])
    ro_total = ro if ro_total is None else ro_total + ro

  return _tc_mlp(ro_total, W1, W2, bn_gamma, bn_beta)
